# baseline (speedup 1.0000x reference)
import functools

import jax
import jax.numpy as jnp
from jax import lax
from jax.experimental import pallas as pl
from jax.experimental.pallas import tpu as pltpu
from jax.experimental.pallas import tpu_sc as plsc

N_ROWS = 1024
N_CLS = 100000
SMOOTHING = 0.1
CONFIDENCE = 1.0 - SMOOTHING

R_CLS = 1024
NB_TC = 75
C0_SC = NB_TC * R_CLS
S_SC = N_CLS - C0_SC
NC, NS, L = 2, 16, 16
S_PC = S_SC // NC
CC = 400
N_CHUNKS = S_PC // CC
CH = CC // 2
FS = 2
CPF = CC // FS
NV = N_ROWS // (8 * L)


def _tc_kernel(x_ref, t_ref, se_ref, sx_ref, xt_ref, s_acc, x_acc, t_acc):
    b = pl.program_id(0)
    nb = pl.num_programs(0)
    t = t_ref[...]

    @pl.when(b == 0)
    def _init():
        s_acc[...] = jnp.zeros_like(s_acc)
        x_acc[...] = jnp.zeros_like(x_acc)
        t_acc[...] = jnp.zeros_like(t_acc)

    x = x_ref[...]
    rows = b * R_CLS + jax.lax.broadcasted_iota(jnp.int32, x.shape, 0)
    e = jnp.exp(jnp.minimum(x, 60.0))
    s_acc[...] += jnp.sum(e, axis=0, keepdims=True)
    x_acc[...] += jnp.sum(x, axis=0, keepdims=True)
    t_acc[...] += jnp.sum(jnp.where(rows == t, x, 0.0), axis=0, keepdims=True)

    @pl.when(b == nb - 1)
    def _fin():
        se_ref[...] = s_acc[...]
        sx_ref[...] = x_acc[...]
        xt_ref[...] = t_acc[...]


def _tc_main(xT, t2d):
    return pl.pallas_call(
        _tc_kernel,
        grid=(NB_TC,),
        in_specs=[
            pl.BlockSpec((R_CLS, N_ROWS), lambda b: (b, 0)),
            pl.BlockSpec((1, N_ROWS), lambda b: (0, 0)),
        ],
        out_specs=[
            pl.BlockSpec((1, N_ROWS), lambda b: (0, 0)),
            pl.BlockSpec((1, N_ROWS), lambda b: (0, 0)),
            pl.BlockSpec((1, N_ROWS), lambda b: (0, 0)),
        ],
        out_shape=[jax.ShapeDtypeStruct((1, N_ROWS), jnp.float32)] * 3,
        scratch_shapes=[pltpu.VMEM((1, N_ROWS), jnp.float32)] * 3,
    )(xT, t2d)


def _sc_body(x_hbm, t_hbm, se_hbm, sx_hbm, xt_hbm,
             sp0, sp1, tbuf, tv, st, semf):
    c = lax.axis_index("c")
    s = lax.axis_index("s")
    h = s // 8
    p = s % 8
    sps = (sp0, sp1)
    zeros = jnp.zeros((L,), jnp.float32)

    pltpu.sync_copy(t_hbm.at[0], tv)
    tks = [tv[pl.ds(p * 128 + k * L, L)] for k in range(8)]

    def fill_desc(g, spbuf):
        cls0 = C0_SC + c * S_PC + g * CC + s * CPF
        return pltpu.make_async_copy(
            x_hbm.at[pl.ds(cls0, CPF)],
            spbuf.at[pl.ds(s * CPF, CPF)], semf)

    @pl.when(s < FS)
    def _prime():
        fill_desc(0, sp0).start()
        fill_desc(0, sp0).wait()

    plsc.subcore_barrier()

    accs = [zeros] * 24

    for g in range(N_CHUNKS):
        cur = sps[g % 2]
        nxt = sps[(g + 1) % 2]
        if g + 1 < N_CHUNKS:
            @pl.when(s < FS)
            def _start_next(g=g, nxt=nxt):
                fill_desc(g + 1, nxt).start()

        pltpu.sync_copy(cur.at[pl.ds(h * CH, CH), pl.ds(p * 128, 128)], tbuf)

        cls_base = C0_SC + c * S_PC + g * CC + h * CH

        def cls_body(i, cry, cls_base=cls_base):
            cry = list(cry)
            clsg = cls_base + i
            for k in range(8):
                xv = tbuf[i, pl.ds(k * L, L)]
                cry[k] = cry[k] + jnp.exp(jnp.minimum(xv, 60.0))
                cry[8 + k] = cry[8 + k] + xv
                cry[16 + k] = cry[16 + k] + jnp.where(tks[k] == clsg, xv, 0.0)
            return tuple(cry)

        accs = list(lax.fori_loop(0, CH, cls_body, tuple(accs)))

        if g + 1 < N_CHUNKS:
            @pl.when(s < FS)
            def _wait_next(g=g, nxt=nxt):
                fill_desc(g + 1, nxt).wait()

        plsc.subcore_barrier()

    out_row = c * 2 + h
    for name, ref, off in ((0, se_hbm, 0), (1, sx_hbm, 8), (2, xt_hbm, 16)):
        for k in range(8):
            st[pl.ds(k * L, L)] = accs[off + k]
        pltpu.sync_copy(st, ref.at[out_row, pl.ds(p * 128, 128)])


def _sc_main(xT, t2d):
    mesh = plsc.VectorSubcoreMesh(core_axis_name="c", subcore_axis_name="s")
    return pl.kernel(
        _sc_body,
        out_type=(jax.ShapeDtypeStruct((4, N_ROWS), jnp.float32),
                  jax.ShapeDtypeStruct((4, N_ROWS), jnp.float32),
                  jax.ShapeDtypeStruct((4, N_ROWS), jnp.float32)),
        mesh=mesh,
        scratch_types=[
            pltpu.MemorySpace.VMEM_SHARED((CC, N_ROWS), jnp.float32),
            pltpu.MemorySpace.VMEM_SHARED((CC, N_ROWS), jnp.float32),
            pltpu.VMEM((CH, 128), jnp.float32),
            pltpu.VMEM((N_ROWS,), jnp.int32),
            pltpu.VMEM((128,), jnp.float32),
            pltpu.SemaphoreType.DMA,
        ],
    )(xT, t2d)


def _combine_kernel(se_tc, sx_tc, xt_tc, se_sc, sx_sc, xt_sc, out_ref):
    se = se_tc[...][0] + jnp.sum(se_sc[...], axis=0)
    sx = sx_tc[...][0] + jnp.sum(sx_sc[...], axis=0)
    xt = xt_tc[...][0] + jnp.sum(xt_sc[...], axis=0)
    losses = (jnp.log(se) - (SMOOTHING / N_CLS) * sx - CONFIDENCE * xt)
    out_ref[...] = (jnp.sum(losses) * (1.0 / N_ROWS)).reshape(1, 1)


@functools.partial(jax.jit, static_argnames=())
def kernel(inputs, targets):
    xT = inputs.T
    t2d = targets.astype(jnp.int32).reshape(1, N_ROWS)
    se_sc, sx_sc, xt_sc = _sc_main(xT, t2d)
    se_tc, sx_tc, xt_tc = _tc_main(xT, t2d)
    out = pl.pallas_call(
        _combine_kernel,
        out_shape=jax.ShapeDtypeStruct((1, 1), jnp.float32),
    )(se_tc, sx_tc, xt_tc, se_sc, sx_sc, xt_sc)
    return out.reshape(())
